# baseline (speedup 1.0000x reference)
import functools

import jax
import jax.numpy as jnp
from jax.experimental import pallas as pl
from jax.experimental.pallas import tpu as pltpu

_EPS = 1e-12
_KNN = 200
_QBLK = 64
_NCHUNK = 8
_NITER = 30


def _agg_kernel(nb, total, x_ref, y_ref, bankt_ref, labels_ref, out_ref,
                sim_ref):
    c = total // _NCHUNK
    xb = x_ref[...]
    nx = jnp.sqrt(jnp.sum(xb * xb, axis=1, keepdims=True))
    v = xb / (nx + _EPS)

    for i in range(_NCHUNK):
        bch = bankt_ref[:, i * c:(i + 1) * c]
        bn = jnp.sqrt(jnp.sum(bch * bch, axis=0, keepdims=True))
        s = jax.lax.dot_general(v, bch, (((1,), (0,)), ((), ())),
                                precision=jax.lax.Precision.HIGHEST,
                                preferred_element_type=jnp.float32)
        s = s / (bn + _EPS)
        col = i * c + jax.lax.broadcasted_iota(jnp.int32, (1, c), 1)
        s = jnp.where(col < nb, s, -3.0)
        sim_ref[:, i * c:(i + 1) * c] = s

    lo0 = jnp.full((_QBLK, 1), -1.001, jnp.float32)
    hi0 = jnp.full((_QBLK, 1), 1.001, jnp.float32)

    def body(_, carry):
        lo, hi = carry
        mid = 0.5 * (lo + hi)
        cnt = jnp.zeros((_QBLK, 1), jnp.float32)
        for i in range(_NCHUNK):
            s = sim_ref[:, i * c:(i + 1) * c]
            cnt = cnt + jnp.sum((s > mid).astype(jnp.float32), axis=1,
                                keepdims=True)
        pred = cnt >= float(_KNN)
        return jnp.where(pred, mid, lo), jnp.where(pred, hi, mid)

    _, t = jax.lax.fori_loop(0, _NITER, body, (lo0, hi0))

    yb = y_ref[...]
    n = jnp.zeros((_QBLK, 1), jnp.float32)
    s1 = jnp.zeros((_QBLK, 1), jnp.float32)
    s2 = jnp.zeros((_QBLK, 1), jnp.float32)
    cm = jnp.zeros((_QBLK, 1), jnp.float32)
    for i in range(_NCHUNK):
        s = sim_ref[:, i * c:(i + 1) * c]
        e = jnp.exp(s)
        gt = s > t
        gtf = gt.astype(jnp.float32)
        n = n + jnp.sum(gtf, axis=1, keepdims=True)
        s1 = s1 + jnp.sum(gtf * e, axis=1, keepdims=True)
        match = (labels_ref[:, i * c:(i + 1) * c] == yb)
        mf = match.astype(jnp.float32)
        s2 = s2 + jnp.sum(gtf * mf * e, axis=1, keepdims=True)
        eqf = (s == t).astype(jnp.float32)
        cm = cm + jnp.sum(eqf * mf, axis=1, keepdims=True)

    r = float(_KNN) - n
    et = jnp.exp(t)
    d1 = s1 + r * et
    d2 = s2 + jnp.minimum(r, cm) * et + _EPS
    row = jnp.log(d1) - jnp.log(d2)
    out_ref[...] = jnp.broadcast_to(row, (_QBLK, 128))


def kernel(x, y, bank, bank_labels):
    nq = x.shape[0]
    nb = bank.shape[0]
    total = ((nb + 1023) // 1024) * 1024

    bankt = jnp.transpose(bank)
    bankt = jnp.pad(bankt, ((0, 0), (0, total - nb)))
    labels = jnp.pad(bank_labels.astype(jnp.int32), (0, total - nb),
                     constant_values=-1).reshape(1, total)
    y2 = y.astype(jnp.int32).reshape(nq, 1)

    grid = nq // _QBLK
    out = pl.pallas_call(
        functools.partial(_agg_kernel, nb, total),
        grid=(grid,),
        in_specs=[
            pl.BlockSpec((_QBLK, x.shape[1]), lambda i: (i, 0)),
            pl.BlockSpec((_QBLK, 1), lambda i: (i, 0)),
            pl.BlockSpec((x.shape[1], total), lambda i: (0, 0)),
            pl.BlockSpec((1, total), lambda i: (0, 0)),
        ],
        out_specs=pl.BlockSpec((_QBLK, 128), lambda i: (i, 0)),
        out_shape=jax.ShapeDtypeStruct((nq, 128), jnp.float32),
        scratch_shapes=[pltpu.VMEM((_QBLK, total), jnp.float32)],
        compiler_params=pltpu.CompilerParams(
            dimension_semantics=("parallel",),
            vmem_limit_bytes=120 * 1024 * 1024),
    )(x, y2, bankt, labels)
    return jnp.sum(out[:, 0]) / nq


# speedup vs baseline: 1.1248x; 1.1248x over previous
import functools

import jax
import jax.numpy as jnp
from jax.experimental import pallas as pl
from jax.experimental.pallas import tpu as pltpu

_EPS = 1e-12
_KNN = 200
_QBLK = 64
_NCHUNK = 16
_NITER = 30


def _agg_kernel(nb, total, x_ref, y_ref, bankt_ref, labels_ref, out_ref,
                sim_ref):
    c = total // _NCHUNK
    xb = x_ref[...]
    nx = jnp.sqrt(jnp.sum(xb * xb, axis=1, keepdims=True))
    v = xb / (nx + _EPS)

    for i in range(_NCHUNK):
        bch = bankt_ref[:, i * c:(i + 1) * c]
        bn = jnp.sqrt(jnp.sum(bch * bch, axis=0, keepdims=True))
        s = jax.lax.dot_general(v, bch, (((1,), (0,)), ((), ())),
                                precision=jax.lax.Precision.HIGHEST,
                                preferred_element_type=jnp.float32)
        s = s / (bn + _EPS)
        col = i * c + jax.lax.broadcasted_iota(jnp.int32, (1, c), 1)
        s = jnp.where(col < nb, s, -3.0)
        sim_ref[:, i * c:(i + 1) * c] = s

    lo0 = jnp.full((_QBLK, 1), -1.001, jnp.float32)
    hi0 = jnp.full((_QBLK, 1), 1.001, jnp.float32)

    def body(_, carry):
        lo, hi = carry
        mid = 0.5 * (lo + hi)
        cnt = jnp.zeros((_QBLK, 1), jnp.float32)
        for i in range(_NCHUNK):
            s = sim_ref[:, i * c:(i + 1) * c]
            cnt = cnt + jnp.sum((s > mid).astype(jnp.float32), axis=1,
                                keepdims=True)
        pred = cnt >= float(_KNN)
        return jnp.where(pred, mid, lo), jnp.where(pred, hi, mid)

    _, t = jax.lax.fori_loop(0, _NITER, body, (lo0, hi0))

    yb = y_ref[...]
    n = jnp.zeros((_QBLK, 1), jnp.float32)
    s1 = jnp.zeros((_QBLK, 1), jnp.float32)
    s2 = jnp.zeros((_QBLK, 1), jnp.float32)
    cm = jnp.zeros((_QBLK, 1), jnp.float32)
    for i in range(_NCHUNK):
        s = sim_ref[:, i * c:(i + 1) * c]
        e = jnp.exp(s)
        gt = s > t
        gtf = gt.astype(jnp.float32)
        n = n + jnp.sum(gtf, axis=1, keepdims=True)
        s1 = s1 + jnp.sum(gtf * e, axis=1, keepdims=True)
        match = (labels_ref[:, i * c:(i + 1) * c] == yb)
        mf = match.astype(jnp.float32)
        s2 = s2 + jnp.sum(gtf * mf * e, axis=1, keepdims=True)
        eqf = (s == t).astype(jnp.float32)
        cm = cm + jnp.sum(eqf * mf, axis=1, keepdims=True)

    r = float(_KNN) - n
    et = jnp.exp(t)
    d1 = s1 + r * et
    d2 = s2 + jnp.minimum(r, cm) * et + _EPS
    row = jnp.log(d1) - jnp.log(d2)
    out_ref[...] = jnp.broadcast_to(row, (_QBLK, 128))


def kernel(x, y, bank, bank_labels):
    nq = x.shape[0]
    nb = bank.shape[0]
    total = ((nb + 1023) // 1024) * 1024

    bankt = jnp.transpose(bank)
    bankt = jnp.pad(bankt, ((0, 0), (0, total - nb)))
    labels = jnp.pad(bank_labels.astype(jnp.int32), (0, total - nb),
                     constant_values=-1).reshape(1, total)
    y2 = y.astype(jnp.int32).reshape(nq, 1)

    grid = nq // _QBLK
    out = pl.pallas_call(
        functools.partial(_agg_kernel, nb, total),
        grid=(grid,),
        in_specs=[
            pl.BlockSpec((_QBLK, x.shape[1]), lambda i: (i, 0)),
            pl.BlockSpec((_QBLK, 1), lambda i: (i, 0)),
            pl.BlockSpec((x.shape[1], total), lambda i: (0, 0)),
            pl.BlockSpec((1, total), lambda i: (0, 0)),
        ],
        out_specs=pl.BlockSpec((_QBLK, 128), lambda i: (i, 0)),
        out_shape=jax.ShapeDtypeStruct((nq, 128), jnp.float32),
        scratch_shapes=[pltpu.VMEM((_QBLK, total), jnp.float32)],
        compiler_params=pltpu.CompilerParams(
            dimension_semantics=("parallel",),
            vmem_limit_bytes=120 * 1024 * 1024),
    )(x, y2, bankt, labels)
    return jnp.sum(out[:, 0]) / nq
